# baseline (speedup 1.0000x reference)
import functools

import jax
import jax.numpy as jnp
from jax import lax
from jax.experimental import pallas as pl
from jax.experimental.pallas import tpu as pltpu
from jax.experimental.pallas import tpu_sc as plsc

N_USERS = 1_000_000
N_ITEMS = 1_000_000
EMBED_DIM = 32
BATCH = 16384

NC, NS, L = 2, 16, 16
NW = NC * NS
B_PER_W = BATCH // NW
CHUNK = 128
NCHUNK = B_PER_W // CHUNK

_mesh = plsc.VectorSubcoreMesh(core_axis_name="c", subcore_axis_name="s")


@functools.partial(
    pl.kernel,
    mesh=_mesh,
    out_type=jax.ShapeDtypeStruct((BATCH, EMBED_DIM), jnp.float32),
    compiler_params=pltpu.CompilerParams(use_tc_tiling_on_sc=False),
    scratch_types=[
        pltpu.VMEM((NCHUNK, CHUNK), jnp.int32),
        pltpu.VMEM((NCHUNK, CHUNK), jnp.int32),
        pltpu.VMEM((B_PER_W, EMBED_DIM), jnp.float32),
        pltpu.VMEM((B_PER_W, EMBED_DIM), jnp.float32),
        pltpu.SemaphoreType.DMA,
    ],
)
def _gmf(user_idx_hbm, item_idx_hbm, user_embed_hbm, item_embed_hbm,
         out_hbm, idx_u, idx_i, rows_u, rows_i, sem):
    wid = lax.axis_index("s") * NC + lax.axis_index("c")
    base = wid * B_PER_W

    for j in range(NCHUNK):
        pltpu.sync_copy(user_idx_hbm.at[pl.ds(base + j * CHUNK, CHUNK)],
                        idx_u.at[j])
        pltpu.sync_copy(item_idx_hbm.at[pl.ds(base + j * CHUNK, CHUNK)],
                        idx_i.at[j])

    copies = []
    for j in range(NCHUNK):
        copies.append(pltpu.async_copy(
            user_embed_hbm.at[idx_u.at[j]],
            rows_u.at[pl.ds(j * CHUNK, CHUNK)], sem))
        copies.append(pltpu.async_copy(
            item_embed_hbm.at[idx_i.at[j]],
            rows_i.at[pl.ds(j * CHUNK, CHUNK)], sem))
    for c in copies:
        c.wait()

    def body(r, _):
        u0 = rows_u[r, pl.ds(0, L)]
        v0 = rows_i[r, pl.ds(0, L)]
        rows_u[r, pl.ds(0, L)] = u0 * v0
        u1 = rows_u[r, pl.ds(L, L)]
        v1 = rows_i[r, pl.ds(L, L)]
        rows_u[r, pl.ds(L, L)] = u1 * v1
        return 0

    lax.fori_loop(0, B_PER_W, body, 0)

    pltpu.sync_copy(rows_u, out_hbm.at[pl.ds(base, B_PER_W)])


def kernel(user_idx, item_idx, user_embed, item_embed):
    return _gmf(user_idx, item_idx, user_embed, item_embed)


# speedup vs baseline: 1.5073x; 1.5073x over previous
import functools

import jax
import jax.numpy as jnp
from jax import lax
from jax.experimental import pallas as pl
from jax.experimental.pallas import tpu as pltpu
from jax.experimental.pallas import tpu_sc as plsc

N_ROWS = 1_000_000
EMBED_DIM = 32
BATCH = 16384

NC, NS, L = 2, 16, 16
NW = NC * NS
B_PER_W = BATCH // NW
PASS_ROWS = 256
NPASS = B_PER_W // PASS_ROWS

_mesh = plsc.VectorSubcoreMesh(core_axis_name="c", subcore_axis_name="s")


@functools.partial(
    pl.kernel,
    mesh=_mesh,
    out_type=jax.ShapeDtypeStruct((BATCH, EMBED_DIM), jnp.float32),
    scratch_types=[
        pltpu.VMEM((B_PER_W,), jnp.int32),
        pltpu.VMEM((B_PER_W,), jnp.int32),
        pltpu.VMEM((PASS_ROWS, EMBED_DIM), jnp.float32),
        pltpu.VMEM((PASS_ROWS, EMBED_DIM), jnp.float32),
        pltpu.SemaphoreType.DMA,
        pltpu.SemaphoreType.DMA,
    ],
)
def _gmf(user_idx_hbm, item_idx_hbm, user_embed_hbm, item_embed_hbm,
         out_hbm, idx_uv, idx_iv, rows_u, rows_i, sem_u, sem_i):
    wid = lax.axis_index("s") * NC + lax.axis_index("c")
    base = wid * B_PER_W

    pltpu.sync_copy(user_idx_hbm.at[pl.ds(base, B_PER_W)], idx_uv)
    pltpu.sync_copy(item_idx_hbm.at[pl.ds(base, B_PER_W)], idx_iv)

    for p in range(NPASS):
        off = p * PASS_ROWS

        def fire(k, _):
            uvec = idx_uv[pl.ds(off + k * L, L)]
            ivec = idx_iv[pl.ds(off + k * L, L)]
            for j in range(L):
                pltpu.async_copy(user_embed_hbm.at[pl.ds(uvec[j], 1), :],
                                 rows_u.at[pl.ds(k * L + j, 1), :], sem_u)
                pltpu.async_copy(item_embed_hbm.at[pl.ds(ivec[j], 1), :],
                                 rows_i.at[pl.ds(k * L + j, 1), :], sem_i)
            return 0

        lax.fori_loop(0, PASS_ROWS // L, fire, 0)

        pltpu.make_async_copy(
            user_embed_hbm.at[pl.ds(0, PASS_ROWS), :], rows_u, sem_u).wait()
        pltpu.make_async_copy(
            item_embed_hbm.at[pl.ds(0, PASS_ROWS), :], rows_i, sem_i).wait()

        def mul(r, _):
            a0 = rows_u[r, pl.ds(0, L)]
            b0 = rows_i[r, pl.ds(0, L)]
            rows_u[r, pl.ds(0, L)] = a0 * b0
            a1 = rows_u[r, pl.ds(L, L)]
            b1 = rows_i[r, pl.ds(L, L)]
            rows_u[r, pl.ds(L, L)] = a1 * b1
            return 0

        lax.fori_loop(0, PASS_ROWS, mul, 0)

        pltpu.sync_copy(rows_u, out_hbm.at[pl.ds(base + off, PASS_ROWS), :])


def kernel(user_idx, item_idx, user_embed, item_embed):
    return _gmf(user_idx, item_idx, user_embed, item_embed)
